# baseline (speedup 1.0000x reference)
import functools

import jax
import jax.numpy as jnp
from jax import lax
from jax.experimental import pallas as pl
from jax.experimental.pallas import tpu as pltpu
from jax.experimental.pallas import tpu_sc as plsc

N = 10000
E = 160000
D_FEAT = 128
D_EDGE = 16
DH = 64
STEPS = 3

NC, NS = 2, 16
NW = NC * NS
CH = 128
EP = 163840
CPW = EP // NW
NCHUNK = CPW // CH
NACC = 10240
RPS = NACC // NS

BE = 4096
BN = 2000

def _dot(a, b):
    return jnp.dot(a, b, preferred_element_type=jnp.float32)


@functools.cache
def _build_sc_gather():
    mesh = plsc.VectorSubcoreMesh(core_axis_name="c", subcore_axis_name="s")

    @functools.partial(
        pl.kernel,
        mesh=mesh,
        out_type=(
            jax.ShapeDtypeStruct((EP, 2 * DH), jnp.float32),
            jax.ShapeDtypeStruct((EP, 2 * DH), jnp.float32),
        ),
        scratch_types=[
            pltpu.VMEM((CH,), jnp.int32),
            pltpu.VMEM((CH,), jnp.int32),
            pltpu.VMEM((CH, 2 * DH), jnp.float32),
            pltpu.VMEM((CH, 2 * DH), jnp.float32),
            pltpu.SemaphoreType.DMA,
            pltpu.SemaphoreType.DMA,
        ],
    )
    def k(t_hbm, s_hbm, r_hbm, ps_out, qr_out,
          si_v, ri_v, ps_v, qr_v, sg1, sg2):
        wid = lax.axis_index("s") * jnp.int32(NC) + lax.axis_index("c")
        base = wid * jnp.int32(CPW)

        @pl.loop(0, NCHUNK)
        def _(i):
            off = base + i * jnp.int32(CH)
            pltpu.sync_copy(s_hbm.at[pl.ds(off, CH)], si_v)
            pltpu.sync_copy(r_hbm.at[pl.ds(off, CH)], ri_v)
            cp1 = pltpu.async_copy(t_hbm.at[si_v], ps_v, sg1)
            cp2 = pltpu.async_copy(t_hbm.at[ri_v], qr_v, sg2)
            cp1.wait()
            cp2.wait()
            cp3 = pltpu.async_copy(ps_v, ps_out.at[pl.ds(off, CH)], sg1)
            cp4 = pltpu.async_copy(qr_v, qr_out.at[pl.ds(off, CH)], sg2)
            cp3.wait()
            cp4.wait()

    return k


def _sc_gather(t_tab, sidx, ridx):
    return _build_sc_gather()(t_tab, sidx, ridx)


@functools.cache
def _build_sc_scatter():
    mesh = plsc.VectorSubcoreMesh(core_axis_name="c", subcore_axis_name="s")

    WD = 2 * DH

    @functools.partial(
        pl.kernel,
        mesh=mesh,
        out_type=jax.ShapeDtypeStruct((NC, NACC, WD), jnp.float32),
        scratch_types=[
            pltpu.VMEM((CH,), jnp.int32),
            pltpu.VMEM((CH, WD), jnp.float32),
            pltpu.VMEM_SHARED((NACC, WD), jnp.float32),
        ],
    )
    def k(e_hbm, r2_hbm, z_hbm, out_hbm, ri_v, e_v, acc_sh):
        cid = lax.axis_index("c")
        sid = lax.axis_index("s")
        wid = sid * jnp.int32(NC) + cid
        accbase = sid * jnp.int32(RPS)

        pltpu.sync_copy(z_hbm, acc_sh.at[pl.ds(accbase, RPS)])

        plsc.subcore_barrier()

        @pl.loop(0, NCHUNK)
        def _(i):
            row = wid * jnp.int32(NCHUNK) + i
            off = wid * jnp.int32(CPW) + i * jnp.int32(CH)
            pltpu.sync_copy(r2_hbm.at[row], ri_v)
            pltpu.sync_copy(e_hbm.at[pl.ds(off, CH)], e_v)
            pltpu.sync_copy(e_v, acc_sh.at[ri_v], add=True)

        plsc.subcore_barrier()
        pltpu.sync_copy(acc_sh.at[pl.ds(accbase, RPS)],
                        out_hbm.at[cid, pl.ds(accbase, RPS)])

    return k


def _sc_scatter(edges, ridx):
    zeros = jnp.zeros((RPS, 2 * DH), jnp.float32)
    r2 = ridx.reshape(EP // CH, CH)
    return _build_sc_scatter()(edges, r2, zeros)


def _pq_call(nodes, wpq):
    n, dn = nodes.shape

    def body(x_ref, w_ref, t_ref):
        t_ref[...] = _dot(x_ref[...], w_ref[...])

    return pl.pallas_call(
        body,
        grid=(n // BN,),
        in_specs=[
            pl.BlockSpec((BN, dn), lambda i: (i, 0)),
            pl.BlockSpec((dn, 2 * DH), lambda i: (0, 0)),
        ],
        out_specs=pl.BlockSpec((BN, 2 * DH), lambda i: (i, 0)),
        out_shape=jax.ShapeDtypeStruct((n, 2 * DH), jnp.float32),
    )(nodes, wpq)


def _edge_call(edges, psx, qrx, w0e, b0, w1, b1, w2, b2, ln_s, ln_b):
    de = edges.shape[1]
    has_ln = ln_s is not None

    def body(e_ref, p_ref, q_ref, w0_ref, b0_ref, w1_ref, b1_ref,
             w2_ref, b2_ref, *rest):
        if has_ln:
            ls_ref, lb_ref, o_ref = rest
        else:
            (o_ref,) = rest
        e = e_ref[...]
        if has_ln:
            e = e[:, :DH]
            mu = jnp.mean(e, axis=1, keepdims=True)
            var = jnp.mean((e - mu) ** 2, axis=1, keepdims=True)
            e = (e - mu) * lax.rsqrt(var + 1e-6) * ls_ref[...] + lb_ref[...]
        pq = p_ref[...][:, :DH] + q_ref[...][:, DH:]
        h = _dot(e, w0_ref[...]) + pq + b0_ref[...]
        h = jax.nn.gelu(h)
        h = jax.nn.gelu(_dot(h, w1_ref[...]) + b1_ref[...])
        out = _dot(h, w2_ref[...]) + b2_ref[...]
        o_ref[...] = jnp.concatenate([out, jnp.zeros_like(out)], axis=1)

    args = [edges, psx, qrx, w0e, b0.reshape(1, DH), w1, b1.reshape(1, DH),
            w2, b2.reshape(1, DH)]
    dw = w0e.shape[0]
    in_specs = [
        pl.BlockSpec((BE, de), lambda i: (i, 0)),
        pl.BlockSpec((BE, 2 * DH), lambda i: (i, 0)),
        pl.BlockSpec((BE, 2 * DH), lambda i: (i, 0)),
        pl.BlockSpec((dw, DH), lambda i: (0, 0)),
        pl.BlockSpec((1, DH), lambda i: (0, 0)),
        pl.BlockSpec((DH, DH), lambda i: (0, 0)),
        pl.BlockSpec((1, DH), lambda i: (0, 0)),
        pl.BlockSpec((DH, DH), lambda i: (0, 0)),
        pl.BlockSpec((1, DH), lambda i: (0, 0)),
    ]
    if has_ln:
        args += [ln_s.reshape(1, DH), ln_b.reshape(1, DH)]
        in_specs += [pl.BlockSpec((1, DH), lambda i: (0, 0)),
                     pl.BlockSpec((1, DH), lambda i: (0, 0))]

    return pl.pallas_call(
        body,
        grid=(EP // BE,),
        in_specs=in_specs,
        out_specs=pl.BlockSpec((BE, 2 * DH), lambda i: (i, 0)),
        out_shape=jax.ShapeDtypeStruct((EP, 2 * DH), jnp.float32),
    )(*args)


def _node_call(nodes, acc, wa, wb, b0, w1, b1, w2, b2, ln_s, ln_b, wpq_next):
    n, dn = nodes.shape
    has_pq = wpq_next is not None

    def body(x_ref, a_ref, wa_ref, wb_ref, b0_ref, w1_ref, b1_ref,
             w2_ref, b2_ref, ls_ref, lb_ref, *rest):
        if has_pq:
            wpq_ref, o_ref, t_ref = rest
        else:
            (o_ref,) = rest
        rec = a_ref[0][:, :DH] + a_ref[1][:, :DH]
        h = _dot(x_ref[...], wa_ref[...]) + _dot(rec, wb_ref[...]) + b0_ref[...]
        h = jax.nn.gelu(h)
        h = jax.nn.gelu(_dot(h, w1_ref[...]) + b1_ref[...])
        h = _dot(h, w2_ref[...]) + b2_ref[...]
        mu = jnp.mean(h, axis=1, keepdims=True)
        var = jnp.mean((h - mu) ** 2, axis=1, keepdims=True)
        h = (h - mu) * lax.rsqrt(var + 1e-6) * ls_ref[...] + lb_ref[...]
        o_ref[...] = h
        if has_pq:
            t_ref[...] = _dot(h, wpq_ref[...])

    args = [nodes, acc, wa, wb, b0.reshape(1, DH), w1, b1.reshape(1, DH),
            w2, b2.reshape(1, DH), ln_s.reshape(1, DH), ln_b.reshape(1, DH)]
    in_specs = [
        pl.BlockSpec((BN, dn), lambda i: (i, 0)),
        pl.BlockSpec((NC, BN, 2 * DH), lambda i: (0, i, 0)),
        pl.BlockSpec((dn, DH), lambda i: (0, 0)),
        pl.BlockSpec((DH, DH), lambda i: (0, 0)),
        pl.BlockSpec((1, DH), lambda i: (0, 0)),
        pl.BlockSpec((DH, DH), lambda i: (0, 0)),
        pl.BlockSpec((1, DH), lambda i: (0, 0)),
        pl.BlockSpec((DH, DH), lambda i: (0, 0)),
        pl.BlockSpec((1, DH), lambda i: (0, 0)),
        pl.BlockSpec((1, DH), lambda i: (0, 0)),
        pl.BlockSpec((1, DH), lambda i: (0, 0)),
    ]
    out_specs = [pl.BlockSpec((BN, DH), lambda i: (i, 0))]
    out_shape = [jax.ShapeDtypeStruct((n, DH), jnp.float32)]
    if has_pq:
        args += [wpq_next]
        in_specs += [pl.BlockSpec((DH, 2 * DH), lambda i: (0, 0))]
        out_specs += [pl.BlockSpec((BN, 2 * DH), lambda i: (i, 0))]
        out_shape += [jax.ShapeDtypeStruct((n, 2 * DH), jnp.float32)]

    res = pl.pallas_call(
        body,
        grid=(n // BN,),
        in_specs=in_specs,
        out_specs=out_specs,
        out_shape=out_shape,
    )(*args)
    return res if has_pq else (res[0], None)


def _readout_call(nodes, ws, bs):
    def body(x_ref, w0_ref, b0_ref, w1_ref, b1_ref, w2_ref, b2_ref, o_ref):
        agg = jnp.sum(x_ref[...], axis=0, keepdims=True) * (1.0 / N)
        h = jax.nn.gelu(_dot(agg, w0_ref[...]) + b0_ref[...])
        h = jax.nn.gelu(_dot(h, w1_ref[...]) + b1_ref[...])
        o_ref[...] = _dot(h, w2_ref[...]) + b2_ref[...]

    return pl.pallas_call(
        body,
        out_shape=jax.ShapeDtypeStruct((1, 1), jnp.float32),
    )(nodes, ws[0], bs[0].reshape(1, -1), ws[1], bs[1].reshape(1, -1),
      ws[2], bs[2].reshape(1, 1))


def kernel(x, edge_index, edge_attr, params):
    with jax.enable_x64(False):
        return _kernel_x32(x, edge_index, edge_attr, params)


def _kernel_x32(x, edge_index, edge_attr, params):
    x = x.astype(jnp.float32)
    senders = edge_index[0].astype(jnp.int32)
    receivers = edge_index[1].astype(jnp.int32)
    pad = EP - E
    sidx = jnp.concatenate([senders, jnp.zeros((pad,), jnp.int32)])
    ridx_g = jnp.concatenate([receivers, jnp.zeros((pad,), jnp.int32)])
    ridx_s = jnp.concatenate([receivers, jnp.full((pad,), N, jnp.int32)])
    edges = jnp.concatenate(
        [edge_attr.astype(jnp.float32), jnp.zeros((pad, D_EDGE), jnp.float32)])

    nodes = x
    t_tab = None
    for s in range(STEPS):
        w0 = params["edge_W"][s][0]
        de = D_EDGE if s == 0 else DH
        dn = D_FEAT if s == 0 else DH
        w0e = w0[:de]
        if s == 0:
            t_tab = _pq_call(
                x, jnp.concatenate([w0[de:de + dn], w0[de + dn:]], axis=1))
        psx, qrx = _sc_gather(t_tab, sidx, ridx_g)
        ln_s = params["ln_e_scale"][s - 1] if s > 0 else None
        ln_b = params["ln_e_bias"][s - 1] if s > 0 else None
        edges = _edge_call(
            edges, psx, qrx, w0e,
            params["edge_b"][s][0], params["edge_W"][s][1],
            params["edge_b"][s][1], params["edge_W"][s][2],
            params["edge_b"][s][2], ln_s, ln_b)
        acc = _sc_scatter(edges, ridx_s)
        wn0 = params["node_W"][s][0]
        if s < STEPS - 1:
            w0n = params["edge_W"][s + 1][0]
            wpq_next = jnp.concatenate([w0n[DH:2 * DH], w0n[2 * DH:]], axis=1)
        else:
            wpq_next = None
        nodes, t_tab = _node_call(
            nodes, acc, wn0[:dn], wn0[dn:],
            params["node_b"][s][0], params["node_W"][s][1],
            params["node_b"][s][1], params["node_W"][s][2],
            params["node_b"][s][2],
            params["ln_n_scale"][s], params["ln_n_bias"][s],
            wpq_next)

    out = _readout_call(nodes, params["ro_W"], params["ro_b"])
    return (out.reshape(1), None)


# speedup vs baseline: 1.2520x; 1.2520x over previous
import functools

import jax
import jax.numpy as jnp
from jax import lax
from jax.experimental import pallas as pl
from jax.experimental.pallas import tpu as pltpu
from jax.experimental.pallas import tpu_sc as plsc

N = 10000
E = 160000
D_FEAT = 128
D_EDGE = 16
DH = 64
STEPS = 3

NC, NS = 2, 16
NW = NC * NS
CH = 128
EP = 163840
CPW = EP // NW
NCHUNK = CPW // CH
NACC = 10240
RPS = NACC // NS

BE = 4096
BN = 2000

def _dot(a, b):
    return jnp.dot(a, b, preferred_element_type=jnp.float32)


@functools.cache
def _build_sc_gather():
    mesh = plsc.VectorSubcoreMesh(core_axis_name="c", subcore_axis_name="s")

    @functools.partial(
        pl.kernel,
        mesh=mesh,
        out_type=jax.ShapeDtypeStruct((EP, DH), jnp.float32),
        scratch_types=[
            pltpu.VMEM((CH,), jnp.int32),
            pltpu.VMEM((CH,), jnp.int32),
            pltpu.VMEM((CH,), jnp.int32),
            pltpu.VMEM((CH,), jnp.int32),
            pltpu.VMEM((CH, 2 * DH), jnp.float32),
            pltpu.VMEM((CH, 2 * DH), jnp.float32),
            pltpu.VMEM((CH, 2 * DH), jnp.float32),
            pltpu.VMEM((CH, 2 * DH), jnp.float32),
            pltpu.VMEM((CH, DH), jnp.float32),
            pltpu.VMEM((CH, DH), jnp.float32),
            pltpu.SemaphoreType.DMA,
            pltpu.SemaphoreType.DMA,
        ],
    )
    def k(t_hbm, s_hbm, r_hbm, o_hbm,
          si_a, ri_a, si_b, ri_b, ps_a, qr_a, ps_b, qr_b, o_a, o_b, sga, sgb):
        wid = lax.axis_index("s") * jnp.int32(NC) + lax.axis_index("c")
        base = wid * jnp.int32(CPW)

        def start(cidx, si_v, ri_v, ps_v, qr_v, sem):
            off = base + cidx * jnp.int32(CH)
            pltpu.sync_copy(s_hbm.at[pl.ds(off, CH)], si_v)
            pltpu.sync_copy(r_hbm.at[pl.ds(off, CH)], ri_v)
            pltpu.async_copy(t_hbm.at[si_v], ps_v, sem)
            pltpu.async_copy(t_hbm.at[ri_v], qr_v, sem)

        def finish(cidx, ps_v, qr_v, o_v, sem):
            pltpu.make_async_copy(t_hbm.at[pl.ds(0, CH)], ps_v, sem).wait()
            pltpu.make_async_copy(t_hbm.at[pl.ds(0, CH)], qr_v, sem).wait()

            @pl.loop(0, CH)
            def _(r):
                @pl.loop(0, DH // 16)
                def _(c):
                    cc = c * jnp.int32(16)
                    o_v[r, pl.ds(cc, 16)] = (
                        ps_v[r, pl.ds(cc, 16)]
                        + qr_v[r, pl.ds(cc + jnp.int32(DH), 16)])

            off = base + cidx * jnp.int32(CH)
            pltpu.sync_copy(o_v, o_hbm.at[pl.ds(off, CH)])

        start(jnp.int32(0), si_a, ri_a, ps_a, qr_a, sga)

        @pl.loop(0, NCHUNK // 2)
        def _(j):
            c0 = j * jnp.int32(2)
            start(c0 + jnp.int32(1), si_b, ri_b, ps_b, qr_b, sgb)
            finish(c0, ps_a, qr_a, o_a, sga)

            @pl.when(j < jnp.int32(NCHUNK // 2 - 1))
            def _():
                start(c0 + jnp.int32(2), si_a, ri_a, ps_a, qr_a, sga)

            finish(c0 + jnp.int32(1), ps_b, qr_b, o_b, sgb)

    return k


def _sc_gather(t_tab, sidx, ridx):
    return _build_sc_gather()(t_tab, sidx, ridx)


@functools.cache
def _build_sc_scatter():
    mesh = plsc.VectorSubcoreMesh(core_axis_name="c", subcore_axis_name="s")

    WD = 2 * DH

    @functools.partial(
        pl.kernel,
        mesh=mesh,
        out_type=jax.ShapeDtypeStruct((NC, NACC, WD), jnp.float32),
        scratch_types=[
            pltpu.VMEM((CH,), jnp.int32),
            pltpu.VMEM((CH, WD), jnp.float32),
            pltpu.VMEM_SHARED((NACC, WD), jnp.float32),
        ],
    )
    def k(e_hbm, r2_hbm, z_hbm, out_hbm, ri_v, e_v, acc_sh):
        cid = lax.axis_index("c")
        sid = lax.axis_index("s")
        wid = sid * jnp.int32(NC) + cid
        accbase = sid * jnp.int32(RPS)

        pltpu.sync_copy(z_hbm, acc_sh.at[pl.ds(accbase, RPS)])

        plsc.subcore_barrier()

        @pl.loop(0, NCHUNK)
        def _(i):
            row = wid * jnp.int32(NCHUNK) + i
            off = wid * jnp.int32(CPW) + i * jnp.int32(CH)
            pltpu.sync_copy(r2_hbm.at[row], ri_v)
            pltpu.sync_copy(e_hbm.at[pl.ds(off, CH)], e_v)
            pltpu.sync_copy(e_v, acc_sh.at[ri_v], add=True)

        plsc.subcore_barrier()
        pltpu.sync_copy(acc_sh.at[pl.ds(accbase, RPS)],
                        out_hbm.at[cid, pl.ds(accbase, RPS)])

    return k


def _sc_scatter(edges, ridx):
    zeros = jnp.zeros((RPS, 2 * DH), jnp.float32)
    r2 = ridx.reshape(EP // CH, CH)
    return _build_sc_scatter()(edges, r2, zeros)


def _pq_call(nodes, wpq):
    n, dn = nodes.shape

    def body(x_ref, w_ref, t_ref):
        t_ref[...] = _dot(x_ref[...], w_ref[...])

    return pl.pallas_call(
        body,
        grid=(n // BN,),
        in_specs=[
            pl.BlockSpec((BN, dn), lambda i: (i, 0)),
            pl.BlockSpec((dn, 2 * DH), lambda i: (0, 0)),
        ],
        out_specs=pl.BlockSpec((BN, 2 * DH), lambda i: (i, 0)),
        out_shape=jax.ShapeDtypeStruct((n, 2 * DH), jnp.float32),
    )(nodes, wpq)


def _edge_call(edges, spq, w0e, b0, w1, b1, w2, b2, ln_s, ln_b):
    de = edges.shape[1]
    has_ln = ln_s is not None

    def body(e_ref, p_ref, w0_ref, b0_ref, w1_ref, b1_ref,
             w2_ref, b2_ref, *rest):
        if has_ln:
            ls_ref, lb_ref, o_ref = rest
        else:
            (o_ref,) = rest
        e = e_ref[...]
        if has_ln:
            e = e[:, :DH]
            mu = jnp.mean(e, axis=1, keepdims=True)
            var = jnp.mean((e - mu) ** 2, axis=1, keepdims=True)
            e = (e - mu) * lax.rsqrt(var + 1e-6) * ls_ref[...] + lb_ref[...]
        h = _dot(e, w0_ref[...]) + p_ref[...] + b0_ref[...]
        h = jax.nn.gelu(h)
        h = jax.nn.gelu(_dot(h, w1_ref[...]) + b1_ref[...])
        out = _dot(h, w2_ref[...]) + b2_ref[...]
        o_ref[...] = jnp.concatenate([out, jnp.zeros_like(out)], axis=1)

    args = [edges, spq, w0e, b0.reshape(1, DH), w1, b1.reshape(1, DH),
            w2, b2.reshape(1, DH)]
    dw = w0e.shape[0]
    in_specs = [
        pl.BlockSpec((BE, de), lambda i: (i, 0)),
        pl.BlockSpec((BE, DH), lambda i: (i, 0)),
        pl.BlockSpec((dw, DH), lambda i: (0, 0)),
        pl.BlockSpec((1, DH), lambda i: (0, 0)),
        pl.BlockSpec((DH, DH), lambda i: (0, 0)),
        pl.BlockSpec((1, DH), lambda i: (0, 0)),
        pl.BlockSpec((DH, DH), lambda i: (0, 0)),
        pl.BlockSpec((1, DH), lambda i: (0, 0)),
    ]
    if has_ln:
        args += [ln_s.reshape(1, DH), ln_b.reshape(1, DH)]
        in_specs += [pl.BlockSpec((1, DH), lambda i: (0, 0)),
                     pl.BlockSpec((1, DH), lambda i: (0, 0))]

    return pl.pallas_call(
        body,
        grid=(EP // BE,),
        in_specs=in_specs,
        out_specs=pl.BlockSpec((BE, 2 * DH), lambda i: (i, 0)),
        out_shape=jax.ShapeDtypeStruct((EP, 2 * DH), jnp.float32),
    )(*args)


def _node_call(nodes, acc, wa, wb, b0, w1, b1, w2, b2, ln_s, ln_b, wpq_next):
    n, dn = nodes.shape
    has_pq = wpq_next is not None

    def body(x_ref, a_ref, wa_ref, wb_ref, b0_ref, w1_ref, b1_ref,
             w2_ref, b2_ref, ls_ref, lb_ref, *rest):
        if has_pq:
            wpq_ref, o_ref, t_ref = rest
        else:
            (o_ref,) = rest
        rec = a_ref[0][:, :DH] + a_ref[1][:, :DH]
        h = _dot(x_ref[...], wa_ref[...]) + _dot(rec, wb_ref[...]) + b0_ref[...]
        h = jax.nn.gelu(h)
        h = jax.nn.gelu(_dot(h, w1_ref[...]) + b1_ref[...])
        h = _dot(h, w2_ref[...]) + b2_ref[...]
        mu = jnp.mean(h, axis=1, keepdims=True)
        var = jnp.mean((h - mu) ** 2, axis=1, keepdims=True)
        h = (h - mu) * lax.rsqrt(var + 1e-6) * ls_ref[...] + lb_ref[...]
        o_ref[...] = h
        if has_pq:
            t_ref[...] = _dot(h, wpq_ref[...])

    args = [nodes, acc, wa, wb, b0.reshape(1, DH), w1, b1.reshape(1, DH),
            w2, b2.reshape(1, DH), ln_s.reshape(1, DH), ln_b.reshape(1, DH)]
    in_specs = [
        pl.BlockSpec((BN, dn), lambda i: (i, 0)),
        pl.BlockSpec((NC, BN, 2 * DH), lambda i: (0, i, 0)),
        pl.BlockSpec((dn, DH), lambda i: (0, 0)),
        pl.BlockSpec((DH, DH), lambda i: (0, 0)),
        pl.BlockSpec((1, DH), lambda i: (0, 0)),
        pl.BlockSpec((DH, DH), lambda i: (0, 0)),
        pl.BlockSpec((1, DH), lambda i: (0, 0)),
        pl.BlockSpec((DH, DH), lambda i: (0, 0)),
        pl.BlockSpec((1, DH), lambda i: (0, 0)),
        pl.BlockSpec((1, DH), lambda i: (0, 0)),
        pl.BlockSpec((1, DH), lambda i: (0, 0)),
    ]
    out_specs = [pl.BlockSpec((BN, DH), lambda i: (i, 0))]
    out_shape = [jax.ShapeDtypeStruct((n, DH), jnp.float32)]
    if has_pq:
        args += [wpq_next]
        in_specs += [pl.BlockSpec((DH, 2 * DH), lambda i: (0, 0))]
        out_specs += [pl.BlockSpec((BN, 2 * DH), lambda i: (i, 0))]
        out_shape += [jax.ShapeDtypeStruct((n, 2 * DH), jnp.float32)]

    res = pl.pallas_call(
        body,
        grid=(n // BN,),
        in_specs=in_specs,
        out_specs=out_specs,
        out_shape=out_shape,
    )(*args)
    return res if has_pq else (res[0], None)


def _readout_call(nodes, ws, bs):
    def body(x_ref, w0_ref, b0_ref, w1_ref, b1_ref, w2_ref, b2_ref, o_ref):
        agg = jnp.sum(x_ref[...], axis=0, keepdims=True) * (1.0 / N)
        h = jax.nn.gelu(_dot(agg, w0_ref[...]) + b0_ref[...])
        h = jax.nn.gelu(_dot(h, w1_ref[...]) + b1_ref[...])
        o_ref[...] = _dot(h, w2_ref[...]) + b2_ref[...]

    return pl.pallas_call(
        body,
        out_shape=jax.ShapeDtypeStruct((1, 1), jnp.float32),
    )(nodes, ws[0], bs[0].reshape(1, -1), ws[1], bs[1].reshape(1, -1),
      ws[2], bs[2].reshape(1, 1))


def kernel(x, edge_index, edge_attr, params):
    with jax.enable_x64(False):
        return _kernel_x32(x, edge_index, edge_attr, params)


def _kernel_x32(x, edge_index, edge_attr, params):
    x = x.astype(jnp.float32)
    senders = edge_index[0].astype(jnp.int32)
    receivers = edge_index[1].astype(jnp.int32)
    pad = EP - E
    sidx = jnp.concatenate([senders, jnp.zeros((pad,), jnp.int32)])
    ridx_g = jnp.concatenate([receivers, jnp.zeros((pad,), jnp.int32)])
    ridx_s = jnp.concatenate([receivers, jnp.full((pad,), N, jnp.int32)])
    edges = jnp.concatenate(
        [edge_attr.astype(jnp.float32), jnp.zeros((pad, D_EDGE), jnp.float32)])

    nodes = x
    t_tab = None
    for s in range(STEPS):
        w0 = params["edge_W"][s][0]
        de = D_EDGE if s == 0 else DH
        dn = D_FEAT if s == 0 else DH
        w0e = w0[:de]
        if s == 0:
            t_tab = _pq_call(
                x, jnp.concatenate([w0[de:de + dn], w0[de + dn:]], axis=1))
        spq = _sc_gather(t_tab, sidx, ridx_g)
        ln_s = params["ln_e_scale"][s - 1] if s > 0 else None
        ln_b = params["ln_e_bias"][s - 1] if s > 0 else None
        edges = _edge_call(
            edges, spq, w0e,
            params["edge_b"][s][0], params["edge_W"][s][1],
            params["edge_b"][s][1], params["edge_W"][s][2],
            params["edge_b"][s][2], ln_s, ln_b)
        acc = _sc_scatter(edges, ridx_s)
        wn0 = params["node_W"][s][0]
        if s < STEPS - 1:
            w0n = params["edge_W"][s + 1][0]
            wpq_next = jnp.concatenate([w0n[DH:2 * DH], w0n[2 * DH:]], axis=1)
        else:
            wpq_next = None
        nodes, t_tab = _node_call(
            nodes, acc, wn0[:dn], wn0[dn:],
            params["node_b"][s][0], params["node_W"][s][1],
            params["node_b"][s][1], params["node_W"][s][2],
            params["node_b"][s][2],
            params["ln_n_scale"][s], params["ln_n_bias"][s],
            wpq_next)

    out = _readout_call(nodes, params["ro_W"], params["ro_b"])
    return (out.reshape(1), None)
